# Initial kernel scaffold; baseline (speedup 1.0000x reference)
import functools

import jax
import jax.numpy as jnp
from jax import lax
from jax.experimental import pallas as pl
from jax.experimental.pallas import tpu as pltpu
from jax.experimental.pallas import tpu_sc as plsc

_NC = 2
_NS = 16
_NW = _NC * _NS
_LANES = 16


def _bag_mean_sc(text2d, emb_weight):
    B, L = text2d.shape
    D = emb_weight.shape[1]
    bags_w = B // _NW
    mesh = plsc.VectorSubcoreMesh(core_axis_name="c", subcore_axis_name="s")

    @functools.partial(
        pl.kernel,
        out_type=jax.ShapeDtypeStruct((B, D), jnp.float32),
        mesh=mesh,
        scratch_types=[
            pltpu.VMEM((bags_w, L), jnp.int32),
            pltpu.VMEM((L, D), jnp.float32),
            pltpu.VMEM((L, D), jnp.float32),
            pltpu.VMEM((bags_w, D), jnp.float32),
            pltpu.SemaphoreType.DMA,
            pltpu.SemaphoreType.DMA,
        ],
    )
    def k(emb_hbm, text_hbm, out_hbm, idx_v, rows0, rows1, out_v, sem0, sem1):
        wid = lax.axis_index("s") * _NC + lax.axis_index("c")
        row0 = wid * bags_w
        pltpu.sync_copy(text_hbm.at[pl.ds(row0, bags_w)], idx_v)

        scale = 1.0 / float(L)

        def accum(rows_ref, b):
            for d in range(D // _LANES):
                sl = pl.ds(d * _LANES, _LANES)
                acc = rows_ref[0, sl]
                for i in range(1, L):
                    acc = acc + rows_ref[i, sl]
                out_v[b, sl] = acc * scale

        def body(g, carry):
            b0 = 2 * g
            b1 = b0 + 1
            c0 = pltpu.async_copy(emb_hbm.at[idx_v.at[b0]], rows0, sem0)
            c1 = pltpu.async_copy(emb_hbm.at[idx_v.at[b1]], rows1, sem1)
            c0.wait()
            accum(rows0, b0)
            c1.wait()
            accum(rows1, b1)
            return carry

        lax.fori_loop(0, bags_w // 2, body, 0)
        pltpu.sync_copy(out_v, out_hbm.at[pl.ds(row0, bags_w)])

    return k(emb_weight, text2d)


def _classifier_tc(bag, fc_weight, fc_bias2d):
    B, D = bag.shape
    C = fc_weight.shape[0]
    blk = 512

    def body(x_ref, w_ref, b_ref, o_ref):
        x = x_ref[...]
        w = w_ref[...]
        logits = lax.dot_general(
            x, w, (((1,), (1,)), ((), ())), preferred_element_type=jnp.float32
        )
        logits = logits + b_ref[...]
        m = jnp.max(logits, axis=1, keepdims=True)
        e = jnp.exp(logits - m)
        lse = jnp.log(jnp.sum(e, axis=1, keepdims=True)) + m
        o_ref[...] = logits - lse

    return pl.pallas_call(
        body,
        grid=(B // blk,),
        in_specs=[
            pl.BlockSpec((blk, D), lambda i: (i, 0)),
            pl.BlockSpec((C, D), lambda i: (0, 0)),
            pl.BlockSpec((1, C), lambda i: (0, 0)),
        ],
        out_specs=pl.BlockSpec((blk, C), lambda i: (i, 0)),
        out_shape=jax.ShapeDtypeStruct((B, C), jnp.float32),
    )(bag, fc_weight, fc_bias2d)


def kernel(text, offsets, emb_weight, fc_weight, fc_bias):
    B = offsets.shape[0]
    T = text.shape[0]
    L = T // B
    C = fc_weight.shape[0]
    text2d = text.reshape(B, L)
    bag = _bag_mean_sc(text2d, emb_weight)
    return _classifier_tc(bag, fc_weight, fc_bias.reshape(1, C))


# speedup vs baseline: 30.0676x; 30.0676x over previous
import functools

import jax
import jax.numpy as jnp
from jax import lax
from jax.experimental import pallas as pl
from jax.experimental.pallas import tpu as pltpu
from jax.experimental.pallas import tpu_sc as plsc

_NC = 2
_NS = 16
_NW = _NC * _NS
_LANES = 16


def _bag_mean_sc(text2d, emb_weight):
    B, L = text2d.shape
    D = emb_weight.shape[1]
    bags_w = B // _NW
    mesh = plsc.VectorSubcoreMesh(core_axis_name="c", subcore_axis_name="s")

    @functools.partial(
        pl.kernel,
        out_type=jax.ShapeDtypeStruct((B, D), jnp.float32),
        mesh=mesh,
        compiler_params=pltpu.CompilerParams(use_tc_tiling_on_sc=False),
        scratch_types=[
            pltpu.VMEM((bags_w, L), jnp.int32),
            pltpu.VMEM((L, D), jnp.float32),
            pltpu.VMEM((L, D), jnp.float32),
            pltpu.VMEM((bags_w, D), jnp.float32),
            pltpu.SemaphoreType.DMA,
            pltpu.SemaphoreType.DMA,
        ],
    )
    def k(emb_hbm, text_hbm, out_hbm, idx_v, rows0, rows1, out_v, sem0, sem1):
        wid = lax.axis_index("s") * _NC + lax.axis_index("c")
        row0 = wid * bags_w
        pltpu.sync_copy(text_hbm.at[pl.ds(row0, bags_w)], idx_v)

        scale = 1.0 / float(L)

        def accum(rows_ref, b):
            for d in range(D // _LANES):
                sl = pl.ds(d * _LANES, _LANES)
                acc = rows_ref[0, sl]
                for i in range(1, L):
                    acc = acc + rows_ref[i, sl]
                out_v[b, sl] = acc * scale

        def body(g, carry):
            b0 = 2 * g
            b1 = b0 + 1
            c0 = pltpu.async_copy(emb_hbm.at[idx_v.at[b0]], rows0, sem0)
            c1 = pltpu.async_copy(emb_hbm.at[idx_v.at[b1]], rows1, sem1)
            c0.wait()
            accum(rows0, b0)
            c1.wait()
            accum(rows1, b1)
            return carry

        lax.fori_loop(0, bags_w // 2, body, 0)
        pltpu.sync_copy(out_v, out_hbm.at[pl.ds(row0, bags_w)])

    return k(emb_weight, text2d)


def _classifier_tc(bag, fc_weight, fc_bias2d):
    B, D = bag.shape
    C = fc_weight.shape[0]
    blk = 512

    def body(x_ref, w_ref, b_ref, o_ref):
        x = x_ref[...]
        w = w_ref[...]
        logits = lax.dot_general(
            x, w, (((1,), (1,)), ((), ())), preferred_element_type=jnp.float32
        )
        logits = logits + b_ref[...]
        m = jnp.max(logits, axis=1, keepdims=True)
        e = jnp.exp(logits - m)
        lse = jnp.log(jnp.sum(e, axis=1, keepdims=True)) + m
        o_ref[...] = logits - lse

    return pl.pallas_call(
        body,
        grid=(B // blk,),
        in_specs=[
            pl.BlockSpec((blk, D), lambda i: (i, 0)),
            pl.BlockSpec((C, D), lambda i: (0, 0)),
            pl.BlockSpec((1, C), lambda i: (0, 0)),
        ],
        out_specs=pl.BlockSpec((blk, C), lambda i: (i, 0)),
        out_shape=jax.ShapeDtypeStruct((B, C), jnp.float32),
    )(bag, fc_weight, fc_bias2d)


def kernel(text, offsets, emb_weight, fc_weight, fc_bias):
    B = offsets.shape[0]
    T = text.shape[0]
    L = T // B
    C = fc_weight.shape[0]
    text2d = text.reshape(B, L)
    bag = _bag_mean_sc(text2d, emb_weight)
    return _classifier_tc(bag, fc_weight, fc_bias.reshape(1, C))
